# Initial kernel scaffold; baseline (speedup 1.0000x reference)
import jax
import jax.numpy as jnp
from jax.experimental import pallas as pl


def kernel(x, edge_index, W1, b1, W2, b2):
    raise NotImplementedError("write your pallas kernel here")


# speedup vs baseline: 57.0295x; 57.0295x over previous
import functools

import jax
import jax.numpy as jnp
from jax import lax
from jax.experimental import pallas as pl
from jax.experimental.pallas import tpu as pltpu
from jax.experimental.pallas import tpu_sc as plsc

N = 10000
IN_DIM = 128
H = 16

NP = 10240
NTILES = 16
CHUNK = 128
ROWS_PER_TILE = 160
E_PAD = NTILES * ROWS_PER_TILE * CHUNK
NSLICE = NP // NTILES


def _mm_body(x_ref, w_ref, o_ref):
    o_ref[...] = jnp.dot(x_ref[...], w_ref[...],
                         preferred_element_type=jnp.float32)


def _matmul(x_pad, W1):
    return pl.pallas_call(
        _mm_body,
        out_shape=jax.ShapeDtypeStruct((NP, H), jnp.float32),
        grid=(NP // 512,),
        in_specs=[
            pl.BlockSpec((512, IN_DIM), lambda i: (i, 0)),
            pl.BlockSpec((IN_DIM, H), lambda i: (0, 0)),
        ],
        out_specs=pl.BlockSpec((512, H), lambda i: (i, 0)),
    )(x_pad, W1)


def _rsqrt16(d):
    ih = plsc.bitcast(d, jnp.int32)
    y = plsc.bitcast(jnp.int32(0x5F3759DF) - (ih >> 1), jnp.float32)
    hd = 0.5 * d
    y = y * (1.5 - hd * y * y)
    y = y * (1.5 - hd * y * y)
    y = y * (1.5 - hd * y * y)
    return y


def _gcn_body(h0_hbm, src_hbm, dst_hbm, b1_hbm, w2_hbm, b2_hbm, out_hbm,
              g_sh, acc1_sh, deg_sh, gs_sh, acc2_sh,
              src_v, dst_v, rows_v, nodes_v, dinv_v, ones_v, svals_v,
              sc_v, out_v, b1_v, w2_v, b2_v):
    sid = lax.axis_index("s")
    nbase = sid * NSLICE
    rbase = sid * ROWS_PER_TILE

    pltpu.sync_copy(src_hbm.at[pl.ds(rbase, ROWS_PER_TILE)], src_v)
    pltpu.sync_copy(dst_hbm.at[pl.ds(rbase, ROWS_PER_TILE)], dst_v)
    pltpu.sync_copy(b1_hbm, b1_v)
    pltpu.sync_copy(w2_hbm, w2_v)
    pltpu.sync_copy(b2_hbm, b2_v)
    for i in range(CHUNK // 16):
        ones_v[pl.ds(i * 16, 16)] = jnp.ones((16,), jnp.float32)

    def _zero(i, c):
        sc_v[pl.ds(i * 16, 16)] = jnp.zeros((16,), jnp.float32)
        return c
    lax.fori_loop(0, NSLICE // 16, _zero, 0)
    pltpu.sync_copy(sc_v, deg_sh.at[pl.ds(nbase, NSLICE)])
    plsc.subcore_barrier()

    def _deg(j, c):
        pltpu.sync_copy(ones_v, deg_sh.at[dst_v.at[j]], add=True)
        return c
    lax.fori_loop(0, ROWS_PER_TILE, _deg, 0)
    plsc.subcore_barrier()

    pltpu.sync_copy(deg_sh.at[pl.ds(nbase, NSLICE)], sc_v)
    def _dinv(i, c):
        d = sc_v[pl.ds(i * 16, 16)] + 1.0
        dinv_v[pl.ds(i * 16, 16)] = _rsqrt16(d)
        return c
    lax.fori_loop(0, NSLICE // 16, _dinv, 0)

    pltpu.sync_copy(h0_hbm.at[pl.ds(nbase, NSLICE)], nodes_v)
    iota = lax.iota(jnp.int32, 16)
    def _scale(gi, c):
        dv = dinv_v[pl.ds(gi * 16, 16)]
        rows = gi * 16 + iota
        for k in range(H):
            cols = jnp.full((16,), k, jnp.int32)
            col = plsc.load_gather(nodes_v, [rows, cols])
            plsc.store_scatter(nodes_v, [rows, cols], col * dv)
        return c
    lax.fori_loop(0, NSLICE // 16, _scale, 0)
    pltpu.sync_copy(nodes_v, g_sh.at[pl.ds(nbase, NSLICE)])
    pltpu.sync_copy(nodes_v, acc1_sh.at[pl.ds(nbase, NSLICE)])
    plsc.subcore_barrier()

    def _l1(j, c):
        pltpu.sync_copy(g_sh.at[src_v.at[j]], rows_v)
        pltpu.sync_copy(rows_v, acc1_sh.at[dst_v.at[j]], add=True)
        return c
    lax.fori_loop(0, ROWS_PER_TILE, _l1, 0)
    plsc.subcore_barrier()

    pltpu.sync_copy(acc1_sh.at[pl.ds(nbase, NSLICE)], nodes_v)
    b1vec = b1_v[...]
    w2vec = w2_v[...]
    def _proj(gi, c):
        dv = dinv_v[pl.ds(gi * 16, 16)]
        acc = jnp.zeros((16,), jnp.float32)
        rows = gi * 16 + iota
        for k in range(H):
            col = plsc.load_gather(
                nodes_v, [rows, jnp.full((16,), k, jnp.int32)])
            hk = jnp.maximum(col * dv + b1vec[k], 0.0)
            acc = acc + hk * w2vec[k]
        sc_v[pl.ds(gi * 16, 16)] = acc * dv
        return c
    lax.fori_loop(0, NSLICE // 16, _proj, 0)
    pltpu.sync_copy(sc_v, gs_sh.at[pl.ds(nbase, NSLICE)])
    pltpu.sync_copy(sc_v, acc2_sh.at[pl.ds(nbase, NSLICE)])
    plsc.subcore_barrier()

    def _l2(j, c):
        pltpu.sync_copy(gs_sh.at[src_v.at[j]], svals_v)
        pltpu.sync_copy(svals_v, acc2_sh.at[dst_v.at[j]], add=True)
        return c
    lax.fori_loop(0, ROWS_PER_TILE, _l2, 0)
    plsc.subcore_barrier()

    pltpu.sync_copy(acc2_sh.at[pl.ds(nbase, NSLICE)], sc_v)
    b2vec = b2_v[...]
    def _out(i, c):
        v = sc_v[pl.ds(i * 16, 16)] * dinv_v[pl.ds(i * 16, 16)] + b2vec
        out_v[pl.ds(i * 16, 16)] = v
        return c
    lax.fori_loop(0, NSLICE // 16, _out, 0)
    pltpu.sync_copy(out_v, out_hbm.at[pl.ds(nbase, NSLICE)])


_gcn_sc = pl.kernel(
    _gcn_body,
    out_type=jax.ShapeDtypeStruct((NP,), jnp.float32),
    mesh=plsc.VectorSubcoreMesh(
        core_axis_name="c", subcore_axis_name="s", num_cores=1),
    scratch_types=[
        pltpu.VMEM_SHARED((NP, H), jnp.float32),
        pltpu.VMEM_SHARED((NP, H), jnp.float32),
        pltpu.VMEM_SHARED((NP,), jnp.float32),
        pltpu.VMEM_SHARED((NP,), jnp.float32),
        pltpu.VMEM_SHARED((NP,), jnp.float32),
        pltpu.VMEM((ROWS_PER_TILE, CHUNK), jnp.int32),
        pltpu.VMEM((ROWS_PER_TILE, CHUNK), jnp.int32),
        pltpu.VMEM((CHUNK, H), jnp.float32),
        pltpu.VMEM((NSLICE, H), jnp.float32),
        pltpu.VMEM((NSLICE,), jnp.float32),
        pltpu.VMEM((CHUNK,), jnp.float32),
        pltpu.VMEM((CHUNK,), jnp.float32),
        pltpu.VMEM((NSLICE,), jnp.float32),
        pltpu.VMEM((NSLICE,), jnp.float32),
        pltpu.VMEM((16,), jnp.float32),
        pltpu.VMEM((16,), jnp.float32),
        pltpu.VMEM((16,), jnp.float32),
    ],
    compiler_params=pltpu.CompilerParams(
        needs_layout_passes=False, use_tc_tiling_on_sc=False),
)


@jax.jit
def kernel(x, edge_index, W1, b1, W2, b2):
    E = edge_index.shape[1]
    x_pad = jnp.concatenate(
        [x, jnp.zeros((NP - N, IN_DIM), jnp.float32)], axis=0)
    h0 = _matmul(x_pad, W1)

    pad = N + (jnp.arange(E_PAD - E, dtype=jnp.int32) % (NP - N))
    srcm = jnp.concatenate([edge_index[0], pad]).reshape(E_PAD // CHUNK, CHUNK)
    dstm = jnp.concatenate([edge_index[1], pad]).reshape(E_PAD // CHUNK, CHUNK)

    b1h = b1.astype(jnp.float32)
    w2h = W2.reshape(H).astype(jnp.float32)
    b2h = jnp.full((16,), b2[0], jnp.float32)

    res = _gcn_sc(h0, srcm, dstm, b1h, w2h, b2h)
    return res[:N]


# speedup vs baseline: 82.0771x; 1.4392x over previous
import jax
import jax.numpy as jnp
from jax import lax
from jax.experimental import pallas as pl
from jax.experimental.pallas import tpu as pltpu
from jax.experimental.pallas import tpu_sc as plsc

N = 10000
IN_DIM = 128
H = 16
E = 320000

NTILES = 16
CHUNK = 80
NCHUNKS = E // (NTILES * CHUNK)
NSLICE = 640
NB = 4


def _mm_body(x_ref, w_ref, o_ref):
    o_ref[...] = jnp.dot(x_ref[...], w_ref[...],
                         preferred_element_type=jnp.float32)


def _matmul(x, W1):
    return pl.pallas_call(
        _mm_body,
        out_shape=jax.ShapeDtypeStruct((N, H), jnp.float32),
        grid=(25,),
        in_specs=[
            pl.BlockSpec((400, IN_DIM), lambda i: (i, 0)),
            pl.BlockSpec((IN_DIM, H), lambda i: (0, 0)),
        ],
        out_specs=pl.BlockSpec((400, H), lambda i: (i, 0)),
    )(x, W1)


def _rsqrt16(d):
    ih = plsc.bitcast(d, jnp.int32)
    y = plsc.bitcast(jnp.int32(0x5F3759DF) - (ih >> 1), jnp.float32)
    hd = 0.5 * d
    y = y * (1.5 - hd * y * y)
    y = y * (1.5 - hd * y * y)
    y = y * (1.5 - hd * y * y)
    return y


def _edge_pass(src_v, dst_v, table_sh, acc_sh, bufs, gsems, ssems):
    n = NCHUNKS
    pltpu.async_copy(table_sh.at[src_v.at[0]], bufs.at[0], gsems.at[0])
    pltpu.async_copy(table_sh.at[src_v.at[1]], bufs.at[1], gsems.at[1])

    def step(j, c):
        b = lax.rem(j, NB)
        bn = lax.rem(j + 2, NB)

        @pl.when(j + 2 < n)
        def _prefetch():
            @pl.when(j >= 2)
            def _drain():
                pltpu.make_async_copy(
                    bufs.at[bn], acc_sh.at[dst_v.at[j - 2]],
                    ssems.at[bn]).wait()
            pltpu.async_copy(
                table_sh.at[src_v.at[j + 2]], bufs.at[bn], gsems.at[bn])

        pltpu.make_async_copy(
            table_sh.at[src_v.at[j]], bufs.at[b], gsems.at[b]).wait()
        pltpu.async_copy(bufs.at[b], acc_sh.at[dst_v.at[j]], ssems.at[b],
                         add=True)
        return c

    lax.fori_loop(0, n, step, 0)
    for t in range(NB):
        j = n - NB + t
        b = j % NB
        pltpu.make_async_copy(
            bufs.at[b], acc_sh.at[dst_v.at[j]], ssems.at[b]).wait()


def _gcn_body(h0_hbm, src_hbm, dst_hbm, b1_hbm, w2_hbm, b2_hbm, out_hbm,
              g_sh, acc1_sh, deg_sh, gs_sh, acc2_sh,
              src_v, dst_v, rows4_v, svals4_v, nodes_v, dinv_v, ones_v,
              sc_v, out_v, b1_v, w2_v, b2_v, dsems, gsems, ssems):
    sid = lax.axis_index("s")
    nbase = lax.min(sid * NSLICE, N - NSLICE)
    rbase = sid * NCHUNKS

    pltpu.sync_copy(src_hbm.at[pl.ds(rbase, NCHUNKS)], src_v)
    pltpu.sync_copy(dst_hbm.at[pl.ds(rbase, NCHUNKS)], dst_v)
    pltpu.sync_copy(b1_hbm, b1_v)
    pltpu.sync_copy(w2_hbm, w2_v)
    pltpu.sync_copy(b2_hbm, b2_v)
    for i in range(CHUNK // 16):
        ones_v[pl.ds(i * 16, 16)] = jnp.ones((16,), jnp.float32)

    def _zero(i, c):
        sc_v[pl.ds(i * 16, 16)] = jnp.zeros((16,), jnp.float32)
        return c
    lax.fori_loop(0, NSLICE // 16, _zero, 0)
    pltpu.sync_copy(sc_v, deg_sh.at[pl.ds(nbase, NSLICE)])
    plsc.subcore_barrier()

    def _deg(j, c):
        b = lax.rem(j, NB)

        @pl.when(j >= NB)
        def _drain():
            pltpu.make_async_copy(
                ones_v, deg_sh.at[dst_v.at[j - NB]], dsems.at[b]).wait()
        pltpu.async_copy(ones_v, deg_sh.at[dst_v.at[j]], dsems.at[b],
                         add=True)
        return c
    lax.fori_loop(0, NCHUNKS, _deg, 0)
    for t in range(NB):
        j = NCHUNKS - NB + t
        pltpu.make_async_copy(
            ones_v, deg_sh.at[dst_v.at[j]], dsems.at[j % NB]).wait()
    plsc.subcore_barrier()

    pltpu.sync_copy(deg_sh.at[pl.ds(nbase, NSLICE)], sc_v)
    def _dinv(i, c):
        d = sc_v[pl.ds(i * 16, 16)] + 1.0
        dinv_v[pl.ds(i * 16, 16)] = _rsqrt16(d)
        return c
    lax.fori_loop(0, NSLICE // 16, _dinv, 0)

    pltpu.sync_copy(h0_hbm.at[pl.ds(nbase, NSLICE)], nodes_v)
    iota = lax.iota(jnp.int32, 16)
    def _scale(gi, c):
        dv = dinv_v[pl.ds(gi * 16, 16)]
        rows = gi * 16 + iota
        for k in range(H):
            cols = jnp.full((16,), k, jnp.int32)
            col = plsc.load_gather(nodes_v, [rows, cols])
            plsc.store_scatter(nodes_v, [rows, cols], col * dv)
        return c
    lax.fori_loop(0, NSLICE // 16, _scale, 0)
    pltpu.sync_copy(nodes_v, g_sh.at[pl.ds(nbase, NSLICE)])
    pltpu.sync_copy(nodes_v, acc1_sh.at[pl.ds(nbase, NSLICE)])
    plsc.subcore_barrier()

    _edge_pass(src_v, dst_v, g_sh, acc1_sh, rows4_v, gsems, ssems)
    plsc.subcore_barrier()

    pltpu.sync_copy(acc1_sh.at[pl.ds(nbase, NSLICE)], nodes_v)
    b1vec = b1_v[...]
    w2vec = w2_v[...]
    def _proj(gi, c):
        dv = dinv_v[pl.ds(gi * 16, 16)]
        acc = jnp.zeros((16,), jnp.float32)
        rows = gi * 16 + iota
        for k in range(H):
            col = plsc.load_gather(
                nodes_v, [rows, jnp.full((16,), k, jnp.int32)])
            hk = jnp.maximum(col * dv + b1vec[k], 0.0)
            acc = acc + hk * w2vec[k]
        sc_v[pl.ds(gi * 16, 16)] = acc * dv
        return c
    lax.fori_loop(0, NSLICE // 16, _proj, 0)
    pltpu.sync_copy(sc_v, gs_sh.at[pl.ds(nbase, NSLICE)])
    pltpu.sync_copy(sc_v, acc2_sh.at[pl.ds(nbase, NSLICE)])
    plsc.subcore_barrier()

    _edge_pass(src_v, dst_v, gs_sh, acc2_sh, svals4_v, gsems, ssems)
    plsc.subcore_barrier()

    pltpu.sync_copy(acc2_sh.at[pl.ds(nbase, NSLICE)], sc_v)
    b2vec = b2_v[...]
    def _out(i, c):
        v = sc_v[pl.ds(i * 16, 16)] * dinv_v[pl.ds(i * 16, 16)] + b2vec
        out_v[pl.ds(i * 16, 16)] = v
        return c
    lax.fori_loop(0, NSLICE // 16, _out, 0)
    pltpu.sync_copy(out_v, out_hbm.at[pl.ds(nbase, NSLICE)])


_gcn_sc = pl.kernel(
    _gcn_body,
    out_type=jax.ShapeDtypeStruct((N,), jnp.float32),
    mesh=plsc.VectorSubcoreMesh(
        core_axis_name="c", subcore_axis_name="s", num_cores=1),
    scratch_types=[
        pltpu.VMEM_SHARED((N, H), jnp.float32),
        pltpu.VMEM_SHARED((N, H), jnp.float32),
        pltpu.VMEM_SHARED((N,), jnp.float32),
        pltpu.VMEM_SHARED((N,), jnp.float32),
        pltpu.VMEM_SHARED((N,), jnp.float32),
        pltpu.VMEM((NCHUNKS, CHUNK), jnp.int32),
        pltpu.VMEM((NCHUNKS, CHUNK), jnp.int32),
        pltpu.VMEM((NB, CHUNK, H), jnp.float32),
        pltpu.VMEM((NB, CHUNK), jnp.float32),
        pltpu.VMEM((NSLICE, H), jnp.float32),
        pltpu.VMEM((NSLICE,), jnp.float32),
        pltpu.VMEM((CHUNK,), jnp.float32),
        pltpu.VMEM((NSLICE,), jnp.float32),
        pltpu.VMEM((NSLICE,), jnp.float32),
        pltpu.VMEM((16,), jnp.float32),
        pltpu.VMEM((16,), jnp.float32),
        pltpu.VMEM((16,), jnp.float32),
        pltpu.SemaphoreType.DMA((NB,)),
        pltpu.SemaphoreType.DMA((NB,)),
        pltpu.SemaphoreType.DMA((NB,)),
    ],
    compiler_params=pltpu.CompilerParams(
        needs_layout_passes=False, use_tc_tiling_on_sc=False),
)


@jax.jit
def kernel(x, edge_index, W1, b1, W2, b2):
    h0 = _matmul(x, W1)
    er = edge_index.reshape(2, E // CHUNK, CHUNK)
    b1h = b1.astype(jnp.float32)
    w2h = W2.reshape(H).astype(jnp.float32)
    b2h = jnp.full((16,), b2[0], jnp.float32)
    return _gcn_sc(h0, er[0], er[1], b1h, w2h, b2h)
